# baseline (speedup 1.0000x reference)
import functools
import math

import jax
import jax.numpy as jnp
import numpy as np
from jax import lax
from jax.experimental import pallas as pl
from jax.experimental.pallas import tpu as pltpu
from jax.experimental.pallas import tpu_sc as plsc

D = 128
SEQ = 200
SCALE = math.sqrt(float(D))


def _positional_encoding(length, depth):
    half = depth // 2
    positions = np.arange(length)[:, None].astype(np.float32)
    depths = np.arange(half)[None, :].astype(np.float32) / float(half)
    angle_rates = 1.0 / (10000.0 ** depths)
    angle_rads = positions * angle_rates
    return np.concatenate([np.sin(angle_rads), np.cos(angle_rads)], axis=-1)


_PE = jnp.asarray(_positional_encoding(2048, D)[:SEQ], dtype=jnp.float32)


@functools.cache
def _make_kernel(n_batch):
    info = plsc.get_sparse_core_info()
    nc, ns = info.num_cores, info.num_subcores
    nw = nc * ns
    seqs_per_w = n_batch // nw
    mesh = plsc.VectorSubcoreMesh(core_axis_name="c", subcore_axis_name="s")

    @functools.partial(
        pl.kernel,
        out_type=jax.ShapeDtypeStruct((n_batch, SEQ, D), jnp.float32),
        mesh=mesh,
        scratch_types=[
            pltpu.VMEM((seqs_per_w * SEQ,), jnp.int32),
            pltpu.VMEM((SEQ, D), jnp.float32),
            pltpu.VMEM((SEQ, D), jnp.float32),
            pltpu.SemaphoreType.DMA,
        ],
    )
    def k(x_hbm, table_hbm, pe_hbm, out_hbm, idx_v, pe_v, rows_v, sem):
        wid = lax.axis_index("s") * nc + lax.axis_index("c")
        base = wid * seqs_per_w * SEQ
        pltpu.sync_copy(x_hbm.at[pl.ds(base, seqs_per_w * SEQ)], idx_v)
        pltpu.sync_copy(pe_hbm, pe_v)

        def seq_body(s, carry):
            pltpu.async_copy(
                table_hbm.at[idx_v.at[pl.ds(s * SEQ, SEQ)]], rows_v, sem
            ).wait()

            def row_body(t, c2):
                for g in range(D // 16):
                    sl = pl.ds(g * 16, 16)
                    rows_v[t, sl] = rows_v[t, sl] * SCALE + pe_v[t, sl]
                return c2

            lax.fori_loop(0, SEQ, row_body, 0)
            pltpu.sync_copy(rows_v, out_hbm.at[wid * seqs_per_w + s])
            return carry

        lax.fori_loop(0, seqs_per_w, seq_body, 0)

    return k


def kernel(x, table):
    n_batch = x.shape[0]
    return _make_kernel(n_batch)(x.reshape(-1), table, _PE)
